# baseline (speedup 1.0000x reference)
import functools
import math

import jax
import jax.numpy as jnp
from jax import lax
from jax.experimental import pallas as pl
from jax.experimental.pallas import tpu as pltpu
from jax.experimental.pallas import tpu_sc as plsc


@functools.lru_cache(maxsize=None)
def _make_gather(B, D, scale):
  info = plsc.get_sparse_core_info()
  NC, NS, L = info.num_cores, info.num_subcores, info.num_lanes
  NW = NC * NS
  assert B % NW == 0 and D % L == 0
  n_per = B // NW
  C = 512
  assert n_per % C == 0
  nch = n_per // C
  mesh = plsc.VectorSubcoreMesh(core_axis_name="c", subcore_axis_name="s")

  @functools.partial(
      pl.kernel,
      mesh=mesh,
      compiler_params=pltpu.CompilerParams(use_tc_tiling_on_sc=False),
      out_type=jax.ShapeDtypeStruct((B, D), jnp.float32),
      scratch_types=[
          pltpu.VMEM((n_per,), jnp.int32),
          pltpu.VMEM((C, D), jnp.float32),
          pltpu.SemaphoreType.DMA,
      ],
  )
  def k(table_hbm, idx_hbm, out_hbm, idx_v, rows_v, sem):
    wid = lax.axis_index("s") * NC + lax.axis_index("c")
    base = pl.multiple_of(wid * n_per, C)
    pltpu.sync_copy(idx_hbm.at[pl.ds(base, n_per)], idx_v)

    def chunk_body(ci, carry):
      off = pl.multiple_of(ci * C, C)
      pltpu.async_copy(
          table_hbm.at[idx_v.at[pl.ds(off, C)]], rows_v, sem
      ).wait()

      def row_body(i, carry2):
        for t in range(D // L):
          sl = pl.ds(t * L, L)
          rows_v[i, sl] = rows_v[i, sl] * scale
        return carry2

      lax.fori_loop(0, C, row_body, 0, unroll=2)
      pltpu.sync_copy(rows_v, out_hbm.at[pl.ds(base + off, C)])
      return carry

    lax.fori_loop(0, nch, chunk_body, 0)

  return k


def kernel(x, table):
  D = table.shape[1]
  B = x.shape[0] * x.shape[1]
  idx = x.reshape(-1).astype(jnp.int32)
  out = _make_gather(B, D, float(math.sqrt(D)))(table, idx)
  return out.reshape(x.shape + (D,))


# speedup vs baseline: 1.0696x; 1.0696x over previous
import functools
import math

import jax
import jax.numpy as jnp
from jax import lax
from jax.experimental import pallas as pl
from jax.experimental.pallas import tpu as pltpu
from jax.experimental.pallas import tpu_sc as plsc

_NBUF = 4


@functools.lru_cache(maxsize=None)
def _make_gather(B, D, scale):
  info = plsc.get_sparse_core_info()
  NC, NS, L = info.num_cores, info.num_subcores, info.num_lanes
  NW = NC * NS
  assert B % NW == 0 and D % L == 0
  n_per = B // NW
  C = 320
  assert n_per % (C * _NBUF) == 0
  nch = n_per // C
  ngrp = nch // _NBUF
  mesh = plsc.VectorSubcoreMesh(core_axis_name="c", subcore_axis_name="s")

  @functools.partial(
      pl.kernel,
      mesh=mesh,
      compiler_params=pltpu.CompilerParams(use_tc_tiling_on_sc=False),
      out_type=jax.ShapeDtypeStruct((B, D), jnp.float32),
      scratch_types=[
          pltpu.VMEM((n_per,), jnp.int32),
          pltpu.VMEM((_NBUF, C, D), jnp.float32),
      ]
      + [pltpu.SemaphoreType.DMA] * (2 * _NBUF),
  )
  def k(table_hbm, idx_hbm, out_hbm, idx_v, rows_v, *sems):
    gsem = sems[:_NBUF]
    wsem = sems[_NBUF:]
    wid = lax.axis_index("s") * NC + lax.axis_index("c")
    base = pl.multiple_of(wid * n_per, C)
    pltpu.sync_copy(idx_hbm.at[pl.ds(base, n_per)], idx_v)

    def start_gather(c, j):
      off = pl.multiple_of(c * C, C)
      pltpu.async_copy(
          table_hbm.at[idx_v.at[pl.ds(off, C)]], rows_v.at[j], gsem[j]
      )

    def wait_gather(j):
      pltpu.make_async_copy(
          out_hbm.at[pl.ds(0, C)], rows_v.at[j], gsem[j]
      ).wait()

    def start_write(c, j):
      off = pl.multiple_of(c * C, C)
      pltpu.async_copy(rows_v.at[j], out_hbm.at[pl.ds(base + off, C)], wsem[j])

    def wait_write(j):
      pltpu.make_async_copy(
          rows_v.at[j], out_hbm.at[pl.ds(base, C)], wsem[j]
      ).wait()

    def scale_buf(j):
      def row_body(i, carry):
        for t in range(D // L):
          sl = pl.ds(t * L, L)
          rows_v[j, i, sl] = rows_v[j, i, sl] * scale
        return carry

      lax.fori_loop(0, C, row_body, 0, unroll=2)

    for j in range(3):
      start_gather(j, j)
    for j in range(_NBUF):
      c = j
      if c + 3 < nch:
        if c >= 1:
          wait_write((j + 3) % _NBUF)
        start_gather(c + 3, (j + 3) % _NBUF)
      wait_gather(j)
      scale_buf(j)
      start_write(c, j)

    def group_body(g, carry):
      c0 = g * _NBUF
      for j in range(_NBUF):
        c = c0 + j
        wait_write((j + 3) % _NBUF)
        start_gather(c + 3, (j + 3) % _NBUF)
        wait_gather(j)
        scale_buf(j)
        start_write(c, j)
      return carry

    lax.fori_loop(1, ngrp - 1, group_body, 0)

    c0 = (ngrp - 1) * _NBUF
    for j in range(_NBUF):
      c = c0 + j
      if c + 3 < nch:
        wait_write((j + 3) % _NBUF)
        start_gather(c + 3, (j + 3) % _NBUF)
      wait_gather(j)
      scale_buf(j)
      start_write(c, j)
    for j in range(_NBUF):
      wait_write(j)

  return k


def kernel(x, table):
  D = table.shape[1]
  B = x.shape[0] * x.shape[1]
  idx = x.reshape(-1).astype(jnp.int32)
  out = _make_gather(B, D, float(math.sqrt(D)))(table, idx)
  return out.reshape(x.shape + (D,))
